# baseline (speedup 1.0000x reference)
import functools

import numpy as np
import jax
import jax.numpy as jnp
from jax import lax
from jax.experimental import pallas as pl
from jax.experimental.pallas import tpu as pltpu
from jax.experimental.pallas import tpu_sc as plsc

N_NODES = 10000
N_EDGES = 320000
D_FEAT = 128
G = 112
NC = 2
NS = 16
NW = NC * NS
GPT = 90
E_PAD = NW * GPT * G
N_PAD = 10112
ROWS_PER_TILE = N_PAD // NS

_PERM = np.zeros(D_FEAT, dtype=np.int32)
for _t in range(D_FEAT // 32):
    for _i in range(16):
        _PERM[32 * _t + 2 * _i] = 32 * _t + _i
        _PERM[32 * _t + 2 * _i + 1] = 32 * _t + 16 + _i


def _sc_partials(feat_bf, colp, rowp, valp, zeros):
    mesh = plsc.VectorSubcoreMesh(core_axis_name="c", subcore_axis_name="s")

    @functools.partial(
        pl.kernel,
        out_type=jax.ShapeDtypeStruct((NC, N_PAD, D_FEAT), jnp.float32),
        mesh=mesh,
        compiler_params=pltpu.CompilerParams(use_tc_tiling_on_sc=False),
        scratch_types=[
            [pltpu.VMEM((G,), jnp.int32) for _ in range(3)],
            [pltpu.VMEM((G,), jnp.int32) for _ in range(3)],
            [pltpu.VMEM((G,), jnp.float32) for _ in range(3)],
            [pltpu.VMEM((G, D_FEAT // 2), jnp.int32) for _ in range(2)],
            [pltpu.VMEM((G, D_FEAT), jnp.float32) for _ in range(2)],
            pltpu.VMEM_SHARED((N_PAD, D_FEAT), jnp.float32),
            [pltpu.SemaphoreType.DMA for _ in range(3)],
            [pltpu.SemaphoreType.DMA for _ in range(3)],
            [pltpu.SemaphoreType.DMA for _ in range(3)],
            [pltpu.SemaphoreType.DMA for _ in range(2)],
            [pltpu.SemaphoreType.DMA for _ in range(3)],
        ],
    )
    def k(feat_hbm, col_hbm, row_hbm, val_hbm, zero_hbm, out_hbm,
          cbuf, rbuf, vbuf, rbf, rfl, acc,
          csem, rsem, vsem, gsem, ssem):
        c = lax.axis_index("c")
        s = lax.axis_index("s")
        wid = s * NC + c
        base = wid * GPT

        r0 = s * ROWS_PER_TILE
        pltpu.sync_copy(zero_hbm.at[pl.ds(r0, ROWS_PER_TILE)],
                        acc.at[pl.ds(r0, ROWS_PER_TILE)])
        plsc.subcore_barrier()

        def c_copy(gi, b):
            return pltpu.make_async_copy(
                col_hbm.at[pl.ds((base + gi) * G, G)], cbuf[b], csem[b])

        def r_copy(gi, b):
            return pltpu.make_async_copy(
                row_hbm.at[pl.ds((base + gi) * G, G)], rbuf[b], rsem[b])

        def v_copy(gi, b):
            return pltpu.make_async_copy(
                val_hbm.at[pl.ds((base + gi) * G, G)], vbuf[b], vsem[b])

        def g_copy(p2, p3):
            return pltpu.make_async_copy(feat_hbm.at[cbuf[p3]],
                                         rbf[p2], gsem[p2])

        def s_copy(p2, p3):
            return pltpu.make_async_copy(rfl[p2], acc.at[rbuf[p3]],
                                         ssem[p3])

        def scale(p2, p3):
            rb = rbf[p2]
            rf = rfl[p2]
            vb = vbuf[p3]

            def t_body(t, _):
                ve = vb[pl.ds(16 * t, 16)]
                for l in range(16):
                    e = 16 * t + l
                    vv = jnp.full((16,), ve[l], jnp.float32)
                    for bt in range(D_FEAT // 32):
                        xi = rb[e, pl.ds(16 * bt, 16)]
                        a0 = lax.bitcast_convert_type(
                            xi << 16, jnp.float32)
                        a1 = lax.bitcast_convert_type(
                            xi & jnp.int32(-65536), jnp.float32)
                        rf[e, pl.ds(32 * bt, 16)] = a0 * vv
                        rf[e, pl.ds(32 * bt + 16, 16)] = a1 * vv
                return 0

            lax.fori_loop(0, G // 16, t_body, 0)

        def slot(ki, p2, p3, ws, w_idx, n1, n2):
            pn3 = (p3 + 1) % 3
            p32 = (p3 + 2) % 3
            if ws:
                s_copy(p2, pn3).wait()
            if n2:
                c_copy(ki + 2, p32).start()
            if n1:
                r_copy(ki + 1, pn3).start()
                v_copy(ki + 1, pn3).start()
                c_copy(ki + 1, pn3).wait()
                g_copy(1 - p2, pn3).start()
            g_copy(p2, p3).wait()
            if w_idx:
                v_copy(ki, p3).wait()
            scale(p2, p3)
            if w_idx:
                r_copy(ki, p3).wait()
            s_copy(p2, p3).start(add=True)

        pltpu.sync_copy(col_hbm.at[pl.ds(base * G, G)], cbuf[0])
        pltpu.sync_copy(row_hbm.at[pl.ds(base * G, G)], rbuf[0])
        pltpu.sync_copy(val_hbm.at[pl.ds(base * G, G)], vbuf[0])
        c_copy(1, 1).start()
        g_copy(0, 0).start()

        slot(0, 0, 0, ws=False, w_idx=False, n1=True, n2=True)
        slot(1, 1, 1, ws=False, w_idx=True, n1=True, n2=True)
        slot(2, 0, 2, ws=True, w_idx=True, n1=True, n2=True)

        def steady(q, _):
            kb = 3 + 6 * q
            for j in range(6):
                slot(kb + j, (1 + j) % 2, j % 3,
                     ws=True, w_idx=True, n1=True, n2=True)
            return 0

        lax.fori_loop(0, (GPT - 6) // 6, steady, 0)

        slot(GPT - 3, 1, 0, ws=True, w_idx=True, n1=True, n2=True)
        slot(GPT - 2, 0, 1, ws=True, w_idx=True, n1=True, n2=False)
        slot(GPT - 1, 1, 2, ws=True, w_idx=True, n1=False, n2=False)
        s_copy(0, 1).wait()
        s_copy(1, 2).wait()

        plsc.subcore_barrier()
        pltpu.sync_copy(acc.at[pl.ds(r0, ROWS_PER_TILE)],
                        out_hbm.at[c, pl.ds(r0, ROWS_PER_TILE)])

    return k(feat_bf, colp, rowp, valp, zeros)


def _combine_kernel(p_ref, o_ref):
    o_ref[...] = p_ref[0] + p_ref[1]


def _combine(partials):
    blk = 1000
    return pl.pallas_call(
        _combine_kernel,
        out_shape=jax.ShapeDtypeStruct((N_NODES, D_FEAT), jnp.float32),
        grid=(N_NODES // blk,),
        in_specs=[pl.BlockSpec((NC, blk, D_FEAT), lambda i: (0, i, 0))],
        out_specs=pl.BlockSpec((blk, D_FEAT), lambda i: (i, 0)),
    )(partials)


def kernel(features, laplacianMat_indices, laplacianMat_values, selfLoop):
    del selfLoop
    pad = E_PAD - N_EDGES
    pad_idx = (jnp.arange(pad, dtype=jnp.int32) % N_NODES)
    rowp = jnp.concatenate([laplacianMat_indices[0], pad_idx])
    colp = jnp.concatenate([laplacianMat_indices[1], pad_idx])
    valp = jnp.concatenate(
        [laplacianMat_values, jnp.zeros((pad,), jnp.float32)])
    feat_bf = features[:, jnp.asarray(_PERM)].astype(jnp.bfloat16)
    feat_i32 = jax.lax.bitcast_convert_type(
        feat_bf.reshape(N_NODES, D_FEAT // 2, 2), jnp.int32)
    zeros = jnp.zeros((N_PAD, D_FEAT), jnp.float32)
    partials = _sc_partials(feat_i32, colp, rowp, valp, zeros)
    return _combine(partials)


# speedup vs baseline: 2.0666x; 2.0666x over previous
import functools

import jax
import jax.numpy as jnp
from jax import lax
from jax.experimental import pallas as pl
from jax.experimental.pallas import tpu as pltpu
from jax.experimental.pallas import tpu_sc as plsc

N_NODES = 10000
N_EDGES = 320000
D_FEAT = 128
G = 112
NC = 2
NS = 16
NW = NC * NS
GPT = 90
E_PAD = NW * GPT * G
N_PAD = 10112
ROWS_PER_TILE = N_PAD // NS
NB = 3


def _sc_partials(features, colp, rowp, valp, zeros):
    mesh = plsc.VectorSubcoreMesh(core_axis_name="c", subcore_axis_name="s")

    @functools.partial(
        pl.kernel,
        out_type=jax.ShapeDtypeStruct((NC, N_PAD, D_FEAT), jnp.float32),
        mesh=mesh,
        scratch_types=[
            [pltpu.VMEM((G,), jnp.int32) for _ in range(NB)],
            [pltpu.VMEM((G,), jnp.int32) for _ in range(NB)],
            [pltpu.VMEM((G,), jnp.float32) for _ in range(NB)],
            [pltpu.VMEM((G, D_FEAT), jnp.float32) for _ in range(NB)],
            pltpu.VMEM_SHARED((N_PAD, D_FEAT), jnp.float32),
            [pltpu.SemaphoreType.DMA for _ in range(NB)],
            [pltpu.SemaphoreType.DMA for _ in range(NB)],
            [pltpu.SemaphoreType.DMA for _ in range(NB)],
            [pltpu.SemaphoreType.DMA for _ in range(NB)],
            [pltpu.SemaphoreType.DMA for _ in range(NB)],
        ],
    )
    def k(feat_hbm, col_hbm, row_hbm, val_hbm, zero_hbm, out_hbm,
          cbuf, rbuf, vbuf, rows, acc,
          csem, rsem, vsem, gsem, ssem):
        c = lax.axis_index("c")
        s = lax.axis_index("s")
        wid = s * NC + c
        base = wid * GPT

        r0 = s * ROWS_PER_TILE
        pltpu.sync_copy(zero_hbm.at[pl.ds(r0, ROWS_PER_TILE)],
                        acc.at[pl.ds(r0, ROWS_PER_TILE)])
        plsc.subcore_barrier()

        def c_copy(gi, b):
            return pltpu.make_async_copy(
                col_hbm.at[pl.ds((base + gi) * G, G)], cbuf[b], csem[b])

        def r_copy(gi, b):
            return pltpu.make_async_copy(
                row_hbm.at[pl.ds((base + gi) * G, G)], rbuf[b], rsem[b])

        def v_copy(gi, b):
            return pltpu.make_async_copy(
                val_hbm.at[pl.ds((base + gi) * G, G)], vbuf[b], vsem[b])

        def g_copy(gi, b):
            del gi
            return pltpu.make_async_copy(feat_hbm.at[cbuf[b]],
                                         rows[b], gsem[b])

        def s_copy(gi, b):
            del gi
            return pltpu.make_async_copy(rows[b], acc.at[rbuf[b]], ssem[b])

        def scale(b):
            rb = rows[b]
            vb = vbuf[b]

            def t_body(t, _):
                ve = vb[pl.ds(16 * t, 16)]
                for l in range(16):
                    e = 16 * t + l
                    vv = jnp.full((16,), ve[l], jnp.float32)
                    a = [rb[e, pl.ds(16 * j, 16)]
                         for j in range(D_FEAT // 16)]
                    for j in range(D_FEAT // 16):
                        rb[e, pl.ds(16 * j, 16)] = a[j] * vv
                return 0

            lax.fori_loop(0, G // 16, t_body, 0)

        def slot(ki, b, ws, w_idx, n1, n2):
            bn = (b + 1) % NB
            b2 = (b + 2) % NB
            if ws:
                s_copy(ki - 2, bn).wait()
            if n2:
                c_copy(ki + 2, b2).start()
            if n1:
                r_copy(ki + 1, bn).start()
                v_copy(ki + 1, bn).start()
                c_copy(ki + 1, bn).wait()
                g_copy(ki + 1, bn).start()
            g_copy(ki, b).wait()
            if w_idx:
                v_copy(ki, b).wait()
            scale(b)
            if w_idx:
                r_copy(ki, b).wait()
            s_copy(ki, b).start(add=True)

        pltpu.sync_copy(col_hbm.at[pl.ds(base * G, G)], cbuf[0])
        pltpu.sync_copy(row_hbm.at[pl.ds(base * G, G)], rbuf[0])
        pltpu.sync_copy(val_hbm.at[pl.ds(base * G, G)], vbuf[0])
        c_copy(1, 1).start()
        g_copy(0, 0).start()

        slot(0, 0, ws=False, w_idx=False, n1=True, n2=True)
        slot(1, 1, ws=False, w_idx=True, n1=True, n2=True)
        slot(2, 2, ws=True, w_idx=True, n1=True, n2=True)

        def steady(q, _):
            for j in range(NB):
                slot(NB * q + j, j, ws=True, w_idx=True, n1=True, n2=True)
            return 0

        lax.fori_loop(1, GPT // NB - 1, steady, 0)

        slot(GPT - 3, 0, ws=True, w_idx=True, n1=True, n2=True)
        slot(GPT - 2, 1, ws=True, w_idx=True, n1=True, n2=False)
        slot(GPT - 1, 2, ws=True, w_idx=True, n1=False, n2=False)
        s_copy(GPT - 2, 1).wait()
        s_copy(GPT - 1, 2).wait()

        plsc.subcore_barrier()
        pltpu.sync_copy(acc.at[pl.ds(r0, ROWS_PER_TILE)],
                        out_hbm.at[c, pl.ds(r0, ROWS_PER_TILE)])

    return k(features, colp, rowp, valp, zeros)


def _combine_kernel(p_ref, o_ref):
    o_ref[...] = p_ref[0] + p_ref[1]


def _combine(partials):
    blk = 1000
    return pl.pallas_call(
        _combine_kernel,
        out_shape=jax.ShapeDtypeStruct((N_NODES, D_FEAT), jnp.float32),
        grid=(N_NODES // blk,),
        in_specs=[pl.BlockSpec((NC, blk, D_FEAT), lambda i: (0, i, 0))],
        out_specs=pl.BlockSpec((blk, D_FEAT), lambda i: (i, 0)),
    )(partials)


def kernel(features, laplacianMat_indices, laplacianMat_values, selfLoop):
    del selfLoop
    pad = E_PAD - N_EDGES
    pad_idx = (jnp.arange(pad, dtype=jnp.int32) % N_NODES)
    rowp = jnp.concatenate([laplacianMat_indices[0], pad_idx])
    colp = jnp.concatenate([laplacianMat_indices[1], pad_idx])
    valp = jnp.concatenate(
        [laplacianMat_values, jnp.zeros((pad,), jnp.float32)])
    zeros = jnp.zeros((N_PAD, D_FEAT), jnp.float32)
    partials = _sc_partials(features, colp, rowp, valp, zeros)
    return _combine(partials)
